# Initial kernel scaffold; baseline (speedup 1.0000x reference)
import jax
import jax.numpy as jnp
from jax.experimental import pallas as pl


def kernel(h0, r, edge_attr, basis_00, W1, b1, g1, be1, W2, b2, g2, be2, W3, b3, edge_index):
    raise NotImplementedError("write your pallas kernel here")


# speedup vs baseline: 1.5597x; 1.5597x over previous
import functools

import jax
import jax.numpy as jnp
from jax import lax
from jax.experimental import pallas as pl
from jax.experimental.pallas import tpu as pltpu
from jax.experimental.pallas import tpu_sc as plsc

_N = 10000
_E = 320000
_M = 16
_MID = 32
_TILE = 2000
_GCHUNK = 2000


def _gather_sc(table, idx):
    info = plsc.get_sparse_core_info()
    nw = info.num_cores * info.num_subcores
    b_per_w = _E // nw
    n_chunks = b_per_w // _GCHUNK
    mesh = plsc.VectorSubcoreMesh(core_axis_name="c", subcore_axis_name="s")

    @functools.partial(
        pl.kernel,
        mesh=mesh,
        out_type=jax.ShapeDtypeStruct((_E, _M), jnp.float32),
        scratch_types=[
            pltpu.VMEM((_GCHUNK,), jnp.int32),
            pltpu.VMEM((_GCHUNK, _M), jnp.float32),
            pltpu.SemaphoreType.DMA,
        ],
        compiler_params=pltpu.CompilerParams(use_tc_tiling_on_sc=False),
    )
    def gather_kernel(table_hbm, idx_hbm, out_hbm, idx_v, rows_v, sem):
        wid = lax.axis_index("s") * info.num_cores + lax.axis_index("c")
        base = wid * b_per_w
        for c in range(n_chunks):
            off = base + c * _GCHUNK
            pltpu.sync_copy(idx_hbm.at[pl.ds(off, _GCHUNK)], idx_v)
            pltpu.async_copy(table_hbm.at[idx_v], rows_v, sem).wait()
            pltpu.sync_copy(rows_v, out_hbm.at[pl.ds(off, _GCHUNK)])

    return gather_kernel(table, idx)


def _tc_body(ea_ref, r_ref, bs_ref, g_ref, w1a_ref, w1r_ref, b1_ref,
             g1_ref, be1_ref, w2_ref, b2_ref, g2_ref, be2_ref, w3_ref,
             b3_ref, tmat_ref, smat_ref, out_ref):
    def ln(y, ga, be):
        mu = jnp.mean(y, axis=-1, keepdims=True)
        var = jnp.mean((y - mu) ** 2, axis=-1, keepdims=True)
        return (y - mu) * lax.rsqrt(var + 1e-5) * ga + be

    y = jnp.dot(ea_ref[...], w1a_ref[...], preferred_element_type=jnp.float32)
    y = y + r_ref[...] * w1r_ref[...] + b1_ref[...]
    y = jnp.maximum(ln(y, g1_ref[...], be1_ref[...]), 0.0)
    y = jnp.dot(y, w2_ref[...], preferred_element_type=jnp.float32) + b2_ref[...]
    y = jnp.maximum(ln(y, g2_ref[...], be2_ref[...]), 0.0)
    y3 = jnp.dot(y, w3_ref[...], preferred_element_type=jnp.float32) + b3_ref[...]
    grep = jnp.dot(g_ref[...], tmat_ref[...], preferred_element_type=jnp.float32)
    out = jnp.dot(y3 * grep, smat_ref[...], preferred_element_type=jnp.float32)
    out_ref[...] = out * bs_ref[...]


def kernel(h0, r, edge_attr, basis_00, W1, b1, g1, be1, W2, b2, g2, be2,
           W3, b3, edge_index):
    table = h0.reshape(_N, _M)
    gathered = _gather_sc(table, edge_index[0])

    c256 = jnp.arange(256)
    tmat = (c256[None, :] % _M == jnp.arange(_M)[:, None]).astype(jnp.float32)
    smat = (c256[:, None] // _M == jnp.arange(_M)[None, :]).astype(jnp.float32)

    w1a = W1[:, :16].T
    w1r = W1[:, 16].reshape(1, _MID)
    bs = basis_00.reshape(_E, 1)

    n_tiles = _E // _TILE
    row_spec = lambda w: pl.BlockSpec((_TILE, w), lambda i: (i, 0))
    full_spec = lambda a: pl.BlockSpec(a.shape, lambda i: (0,) * a.ndim)
    consts = [w1a, w1r, b1.reshape(1, _MID), g1.reshape(1, _MID),
              be1.reshape(1, _MID), W2.T, b2.reshape(1, _MID),
              g2.reshape(1, _MID), be2.reshape(1, _MID), W3.T,
              b3.reshape(1, 256), tmat, smat]

    out = pl.pallas_call(
        _tc_body,
        grid=(n_tiles,),
        in_specs=[row_spec(_M), row_spec(1), row_spec(1), row_spec(_M)]
                 + [full_spec(a) for a in consts],
        out_specs=row_spec(_M),
        out_shape=jax.ShapeDtypeStruct((_E, _M), jnp.float32),
        compiler_params=pltpu.CompilerParams(
            dimension_semantics=("arbitrary",)),
    )(edge_attr, r, bs, gathered, *consts)

    return out.reshape(_E, _M, 1)


# speedup vs baseline: 2.0184x; 1.2942x over previous
import functools

import jax
import jax.numpy as jnp
from jax import lax
from jax.experimental import pallas as pl
from jax.experimental.pallas import tpu as pltpu
from jax.experimental.pallas import tpu_sc as plsc

_N = 10000
_E = 320000
_M = 16
_MID = 32
_P = 8
_EB = _E // _P
_TB = 400
_GCHUNK = 2000


def _gather_sc(table, idx):
    info = plsc.get_sparse_core_info()
    nw = info.num_cores * info.num_subcores
    b_per_w = _E // nw
    n_chunks = b_per_w // _GCHUNK
    mesh = plsc.VectorSubcoreMesh(core_axis_name="c", subcore_axis_name="s")

    @functools.partial(
        pl.kernel,
        mesh=mesh,
        out_type=jax.ShapeDtypeStruct((_E, _M), jnp.float32),
        scratch_types=[
            pltpu.VMEM((_GCHUNK,), jnp.int32),
            pltpu.VMEM((_GCHUNK, _M), jnp.float32),
            pltpu.SemaphoreType.DMA,
        ],
        compiler_params=pltpu.CompilerParams(use_tc_tiling_on_sc=False),
    )
    def gather_kernel(table_hbm, idx_hbm, out_hbm, idx_v, rows_v, sem):
        wid = lax.axis_index("s") * info.num_cores + lax.axis_index("c")
        base = wid * b_per_w
        for c in range(n_chunks):
            off = base + c * _GCHUNK
            pltpu.sync_copy(idx_hbm.at[pl.ds(off, _GCHUNK)], idx_v)
            pltpu.async_copy(table_hbm.at[idx_v], rows_v, sem).wait()
            pltpu.sync_copy(rows_v, out_hbm.at[pl.ds(off, _GCHUNK)])

    return gather_kernel(table, idx)


def _tc_body(ea_ref, r_ref, bs_ref, g_ref, w1a_ref, w1r_ref, b1_ref,
             g1_ref, be1_ref, jm_ref, w2_ref, b2_ref, g2_ref, be2_ref,
             w3_ref, b3_ref, tm_ref, sm_ref, on_ref, out_ref):
    f32 = jnp.float32
    jm = jm_ref[...]

    def ln(y, ga, be):
        mu = jnp.dot(y, jm, preferred_element_type=f32)
        s2 = jnp.dot(y * y, jm, preferred_element_type=f32)
        return (y - mu) * lax.rsqrt(s2 - mu * mu + 1e-5) * ga + be

    y = (jnp.dot(ea_ref[...], w1a_ref[...], preferred_element_type=f32)
         + jnp.dot(r_ref[...], w1r_ref[...], preferred_element_type=f32)
         + b1_ref[...])
    y = jnp.maximum(ln(y, g1_ref[...], be1_ref[...]), 0.0)
    y = jnp.dot(y, w2_ref[...], preferred_element_type=f32) + b2_ref[...]
    y = jnp.maximum(ln(y, g2_ref[...], be2_ref[...]), 0.0)
    y3 = jnp.dot(y, w3_ref[...], preferred_element_type=f32) + b3_ref[...]
    grep = jnp.dot(g_ref[...], tm_ref[...], preferred_element_type=f32)
    out = jnp.dot(y3 * grep, sm_ref[...], preferred_element_type=f32)
    out_ref[...] = out * jnp.dot(bs_ref[...], on_ref[...],
                                 preferred_element_type=f32)


def kernel(h0, r, edge_attr, basis_00, W1, b1, g1, be1, W2, b2, g2, be2,
           W3, b3, edge_index):
    f32 = jnp.float32
    table = h0.reshape(_N, _M)
    gathered = _gather_sc(table, edge_index[0])

    eye = jnp.eye(_P, dtype=f32)

    def kron8(w):
        return jnp.kron(eye, w)

    def tile8(v):
        return jnp.tile(v.reshape(1, -1), (1, _P))

    c256 = jnp.arange(256)
    tmat = (c256[None, :] % _M == jnp.arange(_M)[:, None]).astype(f32)
    smat = (c256[:, None] // _M == jnp.arange(_M)[None, :]).astype(f32)

    consts = [
        kron8(W1[:, :16].T),
        kron8(W1[:, 16].reshape(1, _MID)),
        tile8(b1), tile8(g1), tile8(be1),
        kron8(jnp.full((_MID, _MID), 1.0 / _MID, dtype=f32)),
        kron8(W2.T), tile8(b2), tile8(g2), tile8(be2),
        kron8(W3.T),
        tile8(b3),
        kron8(tmat),
        kron8(smat),
        kron8(jnp.ones((1, _M), dtype=f32)),
    ]

    ea8 = edge_attr.reshape(_EB, _P * _M)
    r8 = r.reshape(_EB, _P)
    bs8 = basis_00.reshape(_EB, _P)
    g8 = gathered.reshape(_EB, _P * _M)

    row_spec = lambda w: pl.BlockSpec((_TB, w), lambda i: (i, 0))
    full_spec = lambda a: pl.BlockSpec(a.shape, lambda i: (0,) * a.ndim)

    out = pl.pallas_call(
        _tc_body,
        grid=(_EB // _TB,),
        in_specs=[row_spec(_P * _M), row_spec(_P), row_spec(_P),
                  row_spec(_P * _M)] + [full_spec(a) for a in consts],
        out_specs=row_spec(_P * _M),
        out_shape=jax.ShapeDtypeStruct((_EB, _P * _M), f32),
        compiler_params=pltpu.CompilerParams(
            dimension_semantics=("arbitrary",)),
    )(ea8, r8, bs8, g8, *consts)

    return out.reshape(_E, _M, 1)


# speedup vs baseline: 2.0268x; 1.0041x over previous
import functools

import jax
import jax.numpy as jnp
from jax import lax
from jax.experimental import pallas as pl
from jax.experimental.pallas import tpu as pltpu
from jax.experimental.pallas import tpu_sc as plsc

_N = 10000
_E = 320000
_M = 16
_MID = 32
_P = 8
_EB = _E // _P
_TB = 400
_GCHUNK = 2000


def _gather_sc(table, idx):
    info = plsc.get_sparse_core_info()
    nw = info.num_cores * info.num_subcores
    b_per_w = _E // nw
    n_chunks = b_per_w // _GCHUNK
    mesh = plsc.VectorSubcoreMesh(core_axis_name="c", subcore_axis_name="s")

    @functools.partial(
        pl.kernel,
        mesh=mesh,
        out_type=jax.ShapeDtypeStruct((_E, _M), jnp.float32),
        scratch_types=[
            pltpu.VMEM((_GCHUNK,), jnp.int32),
            pltpu.VMEM((_GCHUNK, _M), jnp.float32),
            pltpu.SemaphoreType.DMA,
        ],
        compiler_params=pltpu.CompilerParams(use_tc_tiling_on_sc=False),
    )
    def gather_kernel(table_hbm, idx_hbm, out_hbm, idx_v, rows_v, sem):
        wid = lax.axis_index("s") * info.num_cores + lax.axis_index("c")
        base = wid * b_per_w
        for c in range(n_chunks):
            off = base + c * _GCHUNK
            pltpu.sync_copy(idx_hbm.at[pl.ds(off, _GCHUNK)], idx_v)
            pltpu.async_copy(table_hbm.at[idx_v], rows_v, sem).wait()
            pltpu.sync_copy(rows_v, out_hbm.at[pl.ds(off, _GCHUNK)])

    return gather_kernel(table, idx)


def _tc_body(ea_ref, r_ref, bs_ref, g_ref, w1a_ref, w1r_ref, b1_ref,
             g1_ref, be1_ref, jm_ref, w2_ref, b2_ref, g2_ref, be2_ref,
             w3_ref, b3_ref, tm_ref, sm_ref, on_ref, out_ref):
    f32 = jnp.float32
    jm = jm_ref[...]

    def ln(y, ga, be):
        mu = jnp.dot(y, jm, preferred_element_type=f32)
        s2 = jnp.dot(y * y, jm, preferred_element_type=f32)
        return (y - mu) * lax.rsqrt(s2 - mu * mu + 1e-5) * ga + be

    y = (jnp.dot(ea_ref[...], w1a_ref[...], preferred_element_type=f32)
         + jnp.dot(r_ref[...], w1r_ref[...], preferred_element_type=f32)
         + b1_ref[...])
    y = jnp.maximum(ln(y, g1_ref[...], be1_ref[...]), 0.0)
    y = jnp.dot(y, w2_ref[...], preferred_element_type=f32) + b2_ref[...]
    y = jnp.maximum(ln(y, g2_ref[...], be2_ref[...]), 0.0)
    bf = jnp.bfloat16
    y3 = jnp.dot(y.astype(bf), w3_ref[...], preferred_element_type=f32)
    y3 = y3 + b3_ref[...]
    grep = jnp.dot(g_ref[...].astype(bf), tm_ref[...],
                   preferred_element_type=f32)
    out = jnp.dot((y3 * grep).astype(bf), sm_ref[...],
                  preferred_element_type=f32)
    out_ref[...] = out * jnp.dot(bs_ref[...], on_ref[...],
                                 preferred_element_type=f32)


def kernel(h0, r, edge_attr, basis_00, W1, b1, g1, be1, W2, b2, g2, be2,
           W3, b3, edge_index):
    f32 = jnp.float32
    table = h0.reshape(_N, _M)
    gathered = _gather_sc(table, edge_index[0])

    eye = jnp.eye(_P, dtype=f32)

    def kron8(w):
        return jnp.kron(eye, w)

    def tile8(v):
        return jnp.tile(v.reshape(1, -1), (1, _P))

    c256 = jnp.arange(256)
    tmat = (c256[None, :] % _M == jnp.arange(_M)[:, None]).astype(f32)
    smat = (c256[:, None] // _M == jnp.arange(_M)[None, :]).astype(f32)

    consts = [
        kron8(W1[:, :16].T),
        kron8(W1[:, 16].reshape(1, _MID)),
        tile8(b1), tile8(g1), tile8(be1),
        kron8(jnp.full((_MID, _MID), 1.0 / _MID, dtype=f32)),
        kron8(W2.T), tile8(b2), tile8(g2), tile8(be2),
        kron8(W3.T).astype(jnp.bfloat16),
        tile8(b3),
        kron8(tmat).astype(jnp.bfloat16),
        kron8(smat).astype(jnp.bfloat16),
        kron8(jnp.ones((1, _M), dtype=f32)),
    ]

    ea8 = edge_attr.reshape(_EB, _P * _M)
    r8 = r.reshape(_EB, _P)
    bs8 = basis_00.reshape(_EB, _P)
    g8 = gathered.reshape(_EB, _P * _M)

    row_spec = lambda w: pl.BlockSpec((_TB, w), lambda i: (i, 0))
    full_spec = lambda a: pl.BlockSpec(a.shape, lambda i: (0,) * a.ndim)

    out = pl.pallas_call(
        _tc_body,
        grid=(_EB // _TB,),
        in_specs=[row_spec(_P * _M), row_spec(_P), row_spec(_P),
                  row_spec(_P * _M)] + [full_spec(a) for a in consts],
        out_specs=row_spec(_P * _M),
        out_shape=jax.ShapeDtypeStruct((_EB, _P * _M), f32),
        compiler_params=pltpu.CompilerParams(
            dimension_semantics=("arbitrary",)),
    )(ea8, r8, bs8, g8, *consts)

    return out.reshape(_E, _M, 1)


# speedup vs baseline: 2.0885x; 1.0304x over previous
import functools

import jax
import jax.numpy as jnp
from jax import lax
from jax.experimental import pallas as pl
from jax.experimental.pallas import tpu as pltpu
from jax.experimental.pallas import tpu_sc as plsc

_N = 10000
_E = 320000
_M = 16
_MID = 32
_P = 8
_EB = _E // _P
_TB = 800
_GCHUNK = 2000


def _gather_sc(table, idx):
    info = plsc.get_sparse_core_info()
    nw = info.num_cores * info.num_subcores
    b_per_w = _E // nw
    n_chunks = b_per_w // _GCHUNK
    mesh = plsc.VectorSubcoreMesh(core_axis_name="c", subcore_axis_name="s")

    @functools.partial(
        pl.kernel,
        mesh=mesh,
        out_type=jax.ShapeDtypeStruct((_E, _M), jnp.float32),
        scratch_types=[
            pltpu.VMEM((_GCHUNK,), jnp.int32),
            pltpu.VMEM((_GCHUNK, _M), jnp.float32),
            pltpu.SemaphoreType.DMA,
        ],
        compiler_params=pltpu.CompilerParams(use_tc_tiling_on_sc=False),
    )
    def gather_kernel(table_hbm, idx_hbm, out_hbm, idx_v, rows_v, sem):
        wid = lax.axis_index("s") * info.num_cores + lax.axis_index("c")
        base = wid * b_per_w
        for c in range(n_chunks):
            off = base + c * _GCHUNK
            pltpu.sync_copy(idx_hbm.at[pl.ds(off, _GCHUNK)], idx_v)
            pltpu.async_copy(table_hbm.at[idx_v], rows_v, sem).wait()
            pltpu.sync_copy(rows_v, out_hbm.at[pl.ds(off, _GCHUNK)])

    return gather_kernel(table, idx)


def _tc_body(ea_ref, r_ref, bs_ref, g_ref, w1a_ref, w1r_ref, b1_ref,
             g1_ref, be1_ref, jm_ref, w2_ref, b2_ref, g2_ref, be2_ref,
             w3_ref, b3_ref, tm_ref, sm_ref, on_ref, out_ref):
    f32 = jnp.float32
    jm = jm_ref[...]

    def ln(y, ga, be):
        mu = jnp.dot(y, jm, preferred_element_type=f32)
        s2 = jnp.dot(y * y, jm, preferred_element_type=f32)
        return (y - mu) * lax.rsqrt(s2 - mu * mu + 1e-5) * ga + be

    y = (jnp.dot(ea_ref[...], w1a_ref[...], preferred_element_type=f32)
         + jnp.dot(r_ref[...], w1r_ref[...], preferred_element_type=f32)
         + b1_ref[...])
    y = jnp.maximum(ln(y, g1_ref[...], be1_ref[...]), 0.0)
    y = jnp.dot(y, w2_ref[...], preferred_element_type=f32) + b2_ref[...]
    y = jnp.maximum(ln(y, g2_ref[...], be2_ref[...]), 0.0)
    bf = jnp.bfloat16
    y3 = jnp.dot(y.astype(bf), w3_ref[...], preferred_element_type=f32)
    y3 = y3 + b3_ref[...]
    grep = jnp.dot(g_ref[...].astype(bf), tm_ref[...],
                   preferred_element_type=f32)
    out = jnp.dot((y3 * grep).astype(bf), sm_ref[...],
                  preferred_element_type=f32)
    out_ref[...] = out * jnp.dot(bs_ref[...], on_ref[...],
                                 preferred_element_type=f32)


def kernel(h0, r, edge_attr, basis_00, W1, b1, g1, be1, W2, b2, g2, be2,
           W3, b3, edge_index):
    f32 = jnp.float32
    table = h0.reshape(_N, _M)
    gathered = _gather_sc(table, edge_index[0])

    eye = jnp.eye(_P, dtype=f32)

    def kron8(w):
        return jnp.kron(eye, w)

    def tile8(v):
        return jnp.tile(v.reshape(1, -1), (1, _P))

    c256 = jnp.arange(256)
    tmat = (c256[None, :] % _M == jnp.arange(_M)[:, None]).astype(f32)
    smat = (c256[:, None] // _M == jnp.arange(_M)[None, :]).astype(f32)

    consts = [
        kron8(W1[:, :16].T),
        kron8(W1[:, 16].reshape(1, _MID)),
        tile8(b1), tile8(g1), tile8(be1),
        kron8(jnp.full((_MID, _MID), 1.0 / _MID, dtype=f32)),
        kron8(W2.T), tile8(b2), tile8(g2), tile8(be2),
        kron8(W3.T).astype(jnp.bfloat16),
        tile8(b3),
        kron8(tmat).astype(jnp.bfloat16),
        kron8(smat).astype(jnp.bfloat16),
        kron8(jnp.ones((1, _M), dtype=f32)),
    ]

    ea8 = edge_attr.reshape(_EB, _P * _M)
    r8 = r.reshape(_EB, _P)
    bs8 = basis_00.reshape(_EB, _P)
    g8 = gathered.reshape(_EB, _P * _M)

    row_spec = lambda w: pl.BlockSpec((_TB, w), lambda i: (i, 0))
    full_spec = lambda a: pl.BlockSpec(a.shape, lambda i: (0,) * a.ndim)

    out = pl.pallas_call(
        _tc_body,
        grid=(_EB // _TB,),
        in_specs=[row_spec(_P * _M), row_spec(_P), row_spec(_P),
                  row_spec(_P * _M)] + [full_spec(a) for a in consts],
        out_specs=row_spec(_P * _M),
        out_shape=jax.ShapeDtypeStruct((_EB, _P * _M), f32),
        compiler_params=pltpu.CompilerParams(
            dimension_semantics=("arbitrary",)),
    )(ea8, r8, bs8, g8, *consts)

    return out.reshape(_E, _M, 1)


# speedup vs baseline: 2.0908x; 1.0011x over previous
import functools

import jax
import jax.numpy as jnp
from jax import lax
from jax.experimental import pallas as pl
from jax.experimental.pallas import tpu as pltpu
from jax.experimental.pallas import tpu_sc as plsc

_N = 10000
_E = 320000
_M = 16
_MID = 32
_P = 8
_EB = _E // _P
_TB = 800
_GCHUNK = 2000


def _gather_sc(table, idx):
    info = plsc.get_sparse_core_info()
    nw = info.num_cores * info.num_subcores
    b_per_w = _E // nw
    n_chunks = b_per_w // _GCHUNK
    mesh = plsc.VectorSubcoreMesh(core_axis_name="c", subcore_axis_name="s")

    @functools.partial(
        pl.kernel,
        mesh=mesh,
        out_type=jax.ShapeDtypeStruct((_E, _M), jnp.float32),
        scratch_types=[
            pltpu.VMEM((_GCHUNK,), jnp.int32),
            pltpu.VMEM((_GCHUNK, _M), jnp.float32),
            pltpu.SemaphoreType.DMA,
        ],
        compiler_params=pltpu.CompilerParams(use_tc_tiling_on_sc=False),
    )
    def gather_kernel(table_hbm, idx_hbm, out_hbm, idx_v, rows_v, sem):
        wid = lax.axis_index("s") * info.num_cores + lax.axis_index("c")
        base = wid * b_per_w
        for c in range(n_chunks):
            off = base + c * _GCHUNK
            pltpu.sync_copy(idx_hbm.at[pl.ds(off, _GCHUNK)], idx_v)
            pltpu.async_copy(table_hbm.at[idx_v], rows_v, sem).wait()
            pltpu.sync_copy(rows_v, out_hbm.at[pl.ds(off, _GCHUNK)])

    return gather_kernel(table, idx)


def _tc_body(ea_ref, r_ref, bs_ref, g_ref, w1a_ref, w1r_ref, b1_ref,
             g1_ref, be1_ref, jm_ref, w2_ref, b2_ref, g2_ref, be2_ref,
             w3_ref, b3_ref, tm_ref, sm_ref, on_ref, out_ref):
    f32 = jnp.float32
    jm = jm_ref[...]

    def ln(y, ga, be):
        mu = jnp.dot(y, jm, preferred_element_type=f32)
        s2 = jnp.dot(y * y, jm, preferred_element_type=f32)
        return (y - mu) * lax.rsqrt(s2 - mu * mu + 1e-5) * ga + be

    y = (jnp.dot(ea_ref[...], w1a_ref[...], preferred_element_type=f32)
         + jnp.dot(r_ref[...], w1r_ref[...], preferred_element_type=f32)
         + b1_ref[...])
    y = jnp.maximum(ln(y, g1_ref[...], be1_ref[...]), 0.0)
    y = jnp.dot(y, w2_ref[...], preferred_element_type=f32) + b2_ref[...]
    y = jnp.maximum(ln(y, g2_ref[...], be2_ref[...]), 0.0)
    bf = jnp.bfloat16
    y3 = jnp.dot(y.astype(bf), w3_ref[...], preferred_element_type=f32)
    y3 = y3 + b3_ref[...]
    grep = jnp.dot(g_ref[...].astype(bf), tm_ref[...],
                   preferred_element_type=f32)
    out = jnp.dot((y3 * grep).astype(bf), sm_ref[...],
                  preferred_element_type=f32)
    out_ref[...] = out * jnp.dot(bs_ref[...], on_ref[...],
                                 preferred_element_type=f32)


def kernel(h0, r, edge_attr, basis_00, W1, b1, g1, be1, W2, b2, g2, be2,
           W3, b3, edge_index):
    f32 = jnp.float32
    table = h0.reshape(_N, _M)
    gathered = _gather_sc(table, edge_index[0])

    eye = jnp.eye(_P, dtype=f32)

    def kron8(w):
        return jnp.kron(eye, w)

    def tile8(v):
        return jnp.tile(v.reshape(1, -1), (1, _P))

    c256 = jnp.arange(256)
    tmat = (c256[None, :] % _M == jnp.arange(_M)[:, None]).astype(f32)
    smat = (c256[:, None] // _M == jnp.arange(_M)[None, :]).astype(f32)

    consts = [
        kron8(W1[:, :16].T),
        kron8(W1[:, 16].reshape(1, _MID)),
        tile8(b1), tile8(g1), tile8(be1),
        kron8(jnp.full((_MID, _MID), 1.0 / _MID, dtype=f32)),
        kron8(W2.T), tile8(b2), tile8(g2), tile8(be2),
        kron8(W3.T).astype(jnp.bfloat16),
        tile8(b3),
        kron8(tmat).astype(jnp.bfloat16),
        kron8(smat).astype(jnp.bfloat16),
        kron8(jnp.ones((1, _M), dtype=f32)),
    ]

    ea8 = edge_attr.reshape(_EB, _P * _M)
    r8 = r.reshape(_EB, _P)
    bs8 = basis_00.reshape(_EB, _P)
    g8 = gathered.reshape(_EB, _P * _M)

    row_spec = lambda w: pl.BlockSpec((_TB, w), lambda i: (i, 0))
    full_spec = lambda a: pl.BlockSpec(a.shape, lambda i: (0,) * a.ndim)

    out = pl.pallas_call(
        _tc_body,
        grid=(_EB // _TB,),
        in_specs=[row_spec(_P * _M), row_spec(_P), row_spec(_P),
                  row_spec(_P * _M)] + [full_spec(a) for a in consts],
        out_specs=row_spec(_P * _M),
        out_shape=jax.ShapeDtypeStruct((_EB, _P * _M), f32),
        compiler_params=pltpu.CompilerParams(
            dimension_semantics=("arbitrary",),
            allow_input_fusion=[True, True, True, True]
                               + [False] * len(consts)),
    )(ea8, r8, bs8, g8, *consts)

    return out.reshape(_E, _M, 1)


# speedup vs baseline: 3.0990x; 1.4822x over previous
import functools

import jax
import jax.numpy as jnp
from jax import lax
from jax.experimental import pallas as pl
from jax.experimental.pallas import tpu as pltpu
from jax.experimental.pallas import tpu_sc as plsc

_N = 10000
_E = 320000
_M = 16
_MID = 32
_TE = 6400
_GCHUNK = 2000


def _gather_sc(table, idx):
    info = plsc.get_sparse_core_info()
    nw = info.num_cores * info.num_subcores
    b_per_w = _E // nw
    n_chunks = b_per_w // _GCHUNK
    mesh = plsc.VectorSubcoreMesh(core_axis_name="c", subcore_axis_name="s")

    @functools.partial(
        pl.kernel,
        mesh=mesh,
        out_type=jax.ShapeDtypeStruct((_E, _M), jnp.float32),
        scratch_types=[
            pltpu.VMEM((_GCHUNK,), jnp.int32),
            pltpu.VMEM((_GCHUNK, _M), jnp.float32),
            pltpu.SemaphoreType.DMA,
        ],
        compiler_params=pltpu.CompilerParams(use_tc_tiling_on_sc=False),
    )
    def gather_kernel(table_hbm, idx_hbm, out_hbm, idx_v, rows_v, sem):
        wid = lax.axis_index("s") * info.num_cores + lax.axis_index("c")
        base = wid * b_per_w
        for c in range(n_chunks):
            off = base + c * _GCHUNK
            pltpu.sync_copy(idx_hbm.at[pl.ds(off, _GCHUNK)], idx_v)
            pltpu.async_copy(table_hbm.at[idx_v], rows_v, sem).wait()
            pltpu.sync_copy(rows_v, out_hbm.at[pl.ds(off, _GCHUNK)])

    return gather_kernel(table, idx)


def _tc_body(ea_ref, r_ref, bs_ref, g_ref, w1a_ref, w1rb_ref, jm_ref,
             gb1_ref, w2b_ref, gb2_ref, w3_ref, b3_ref, tm_ref, sm_ref,
             out_ref):
    f32 = jnp.float32
    bf = jnp.bfloat16
    jm = jm_ref[...]
    ones_row = jnp.ones((1, _TE), dtype=f32)

    def dotf(a, b):
        return jnp.dot(a, b, preferred_element_type=f32)

    def ln_relu(y, gb):
        mu = dotf(jm, y)
        s2 = dotf(jm, y * y)
        ga = dotf(gb[0:1, :].T, ones_row)
        be = dotf(gb[1:2, :].T, ones_row)
        return jnp.maximum((y - mu) * lax.rsqrt(s2 - mu * mu + 1e-5)
                           * ga + be, 0.0)

    aug = jnp.concatenate([r_ref[...], ones_row], axis=0)
    y = dotf(w1a_ref[...], ea_ref[...]) + dotf(w1rb_ref[...], aug)
    y = ln_relu(y, gb1_ref[...])
    y = dotf(w2b_ref[...][:, :_MID], y) \
        + dotf(w2b_ref[...][:, _MID:], ones_row)
    y = ln_relu(y, gb2_ref[...])
    y3 = dotf(w3_ref[...], y.astype(bf)) + dotf(b3_ref[...], ones_row)
    grep = dotf(tm_ref[...], g_ref[...].astype(bf))
    out = dotf(sm_ref[...], (y3 * grep).astype(bf))
    out_ref[...] = out * dotf(jnp.ones((_M, 1), dtype=f32), bs_ref[...])


def kernel(h0, r, edge_attr, basis_00, W1, b1, g1, be1, W2, b2, g2, be2,
           W3, b3, edge_index):
    f32 = jnp.float32
    table = h0.reshape(_N, _M)
    gathered = _gather_sc(table, edge_index[0])

    eaT = edge_attr.T
    rT = r.T
    bsT = basis_00.reshape(1, _E)
    gT = gathered.T

    eye16 = jnp.eye(_M, dtype=f32)
    consts = [
        W1[:, :16],
        jnp.stack([W1[:, 16], b1], axis=1),
        jnp.full((_MID, _MID), 1.0 / _MID, dtype=f32),
        jnp.stack([g1, be1], axis=0),
        jnp.concatenate([W2, b2[:, None]], axis=1),
        jnp.stack([g2, be2], axis=0),
        W3.astype(jnp.bfloat16),
        b3.reshape(256, 1),
        jnp.tile(eye16, (_M, 1)).astype(jnp.bfloat16),
        jnp.kron(eye16, jnp.ones((1, _M))).astype(jnp.bfloat16),
    ]

    col_spec = lambda h: pl.BlockSpec((h, _TE), lambda i: (0, i))
    full_spec = lambda a: pl.BlockSpec(a.shape, lambda i: (0,) * a.ndim)

    outT = pl.pallas_call(
        _tc_body,
        grid=(_E // _TE,),
        in_specs=[col_spec(_M), col_spec(1), col_spec(1), col_spec(_M)]
                 + [full_spec(a) for a in consts],
        out_specs=col_spec(_M),
        out_shape=jax.ShapeDtypeStruct((_M, _E), f32),
        compiler_params=pltpu.CompilerParams(
            dimension_semantics=("arbitrary",)),
    )(eaT, rT, bsT, gT, *consts)

    return outT.T.reshape(_E, _M, 1)


# speedup vs baseline: 3.3643x; 1.0856x over previous
import functools

import jax
import jax.numpy as jnp
from jax import lax
from jax.experimental import pallas as pl
from jax.experimental.pallas import tpu as pltpu
from jax.experimental.pallas import tpu_sc as plsc

_N = 10000
_E = 320000
_M = 16
_MID = 32
_TE = 6400
_NH = 2
_EH = _E // _NH
_GCHUNK = 5000


def _gather_sc(table, idx, n_edges):
    info = plsc.get_sparse_core_info()
    nw = info.num_cores * info.num_subcores
    b_per_w = n_edges // nw
    n_chunks = b_per_w // _GCHUNK
    mesh = plsc.VectorSubcoreMesh(core_axis_name="c", subcore_axis_name="s")

    @functools.partial(
        pl.kernel,
        mesh=mesh,
        out_type=jax.ShapeDtypeStruct((n_edges, _M), jnp.float32),
        scratch_types=[
            pltpu.VMEM((_GCHUNK,), jnp.int32),
            pltpu.VMEM((_GCHUNK, _M), jnp.float32),
            pltpu.SemaphoreType.DMA,
        ],
        compiler_params=pltpu.CompilerParams(use_tc_tiling_on_sc=False),
    )
    def gather_kernel(table_hbm, idx_hbm, out_hbm, idx_v, rows_v, sem):
        wid = lax.axis_index("s") * info.num_cores + lax.axis_index("c")
        base = wid * b_per_w
        for c in range(n_chunks):
            off = base + c * _GCHUNK
            pltpu.sync_copy(idx_hbm.at[pl.ds(off, _GCHUNK)], idx_v)
            pltpu.async_copy(table_hbm.at[idx_v], rows_v, sem).wait()
            pltpu.sync_copy(rows_v, out_hbm.at[pl.ds(off, _GCHUNK)])

    return gather_kernel(table, idx)


def _tc_body(ea_ref, r_ref, bs_ref, g_ref, w1a_ref, w1rb_ref, jm_ref,
             gb1_ref, w2b_ref, gb2_ref, w3_ref, b3_ref, tm_ref, sm_ref,
             out_ref):
    f32 = jnp.float32
    bf = jnp.bfloat16
    jm = jm_ref[...]
    ones_row = jnp.ones((1, _TE), dtype=f32)

    def dotf(a, b):
        return jnp.dot(a, b, preferred_element_type=f32)

    def ln_relu(y, gb):
        mu = dotf(jm, y)
        s2 = dotf(jm, y * y)
        ga = dotf(gb[0:1, :].T, ones_row)
        be = dotf(gb[1:2, :].T, ones_row)
        return jnp.maximum((y - mu) * lax.rsqrt(s2 - mu * mu + 1e-5)
                           * ga + be, 0.0)

    aug = jnp.concatenate([r_ref[...], ones_row], axis=0)
    y = dotf(w1a_ref[...], ea_ref[...]) + dotf(w1rb_ref[...], aug)
    y = ln_relu(y, gb1_ref[...])
    y = dotf(w2b_ref[...][:, :_MID], y) \
        + dotf(w2b_ref[...][:, _MID:], ones_row)
    y = ln_relu(y, gb2_ref[...])
    y3 = dotf(w3_ref[...], y.astype(bf)) + dotf(b3_ref[...], ones_row)
    grep = dotf(tm_ref[...], g_ref[...].astype(bf))
    out = dotf(sm_ref[...], (y3 * grep).astype(bf))
    out_ref[...] = out * dotf(jnp.ones((_M, 1), dtype=f32), bs_ref[...])


def kernel(h0, r, edge_attr, basis_00, W1, b1, g1, be1, W2, b2, g2, be2,
           W3, b3, edge_index):
    f32 = jnp.float32
    table = h0.reshape(_N, _M)
    src_idx = edge_index[0]

    eaT = edge_attr.T
    rT = r.T
    bsT = basis_00.reshape(1, _E)

    eye16 = jnp.eye(_M, dtype=f32)
    consts = [
        W1[:, :16],
        jnp.stack([W1[:, 16], b1], axis=1),
        jnp.full((_MID, _MID), 1.0 / _MID, dtype=f32),
        jnp.stack([g1, be1], axis=0),
        jnp.concatenate([W2, b2[:, None]], axis=1),
        jnp.stack([g2, be2], axis=0),
        W3.astype(jnp.bfloat16),
        b3.reshape(256, 1),
        jnp.tile(eye16, (_M, 1)).astype(jnp.bfloat16),
        jnp.kron(eye16, jnp.ones((1, _M))).astype(jnp.bfloat16),
    ]

    full_spec = lambda a: pl.BlockSpec(a.shape, lambda i: (0,) * a.ndim)
    nblk = _EH // _TE

    halves = []
    for h in range(_NH):
        idx_h = lax.slice_in_dim(src_idx, h * _EH, (h + 1) * _EH)
        gT_h = _gather_sc(table, idx_h, _EH).T
        off_spec = lambda w, hh=h: pl.BlockSpec(
            (w, _TE), lambda i, _hh=hh: (0, i + _hh * nblk))
        loc_spec = lambda w: pl.BlockSpec((w, _TE), lambda i: (0, i))
        outT_h = pl.pallas_call(
            _tc_body,
            grid=(nblk,),
            in_specs=[off_spec(_M), off_spec(1), off_spec(1), loc_spec(_M)]
                     + [full_spec(a) for a in consts],
            out_specs=loc_spec(_M),
            out_shape=jax.ShapeDtypeStruct((_M, _EH), f32),
            compiler_params=pltpu.CompilerParams(
                dimension_semantics=("arbitrary",)),
        )(eaT, rT, bsT, gT_h, *consts)
        halves.append(outT_h)

    outT = jnp.concatenate(halves, axis=1)
    return outT.T.reshape(_E, _M, 1)
